# baseline (speedup 1.0000x reference)
import functools

import jax
import jax.numpy as jnp
from jax import lax
from jax.experimental import pallas as pl
from jax.experimental.pallas import tpu as pltpu
from jax.experimental.pallas import tpu_sc as plsc

HIDDEN = 1024
N_PASSES = 3
LANES = 16
N_WORKERS = 32
TILE_ROWS = 32
TILE_ELEMS = TILE_ROWS * HIDDEN

TC_ROWS = 27648
TC_A_ROWS = 11264
TC_BLOCK_ROWS = 1024


def _col_params(angles, plane_i, plane_j):
    cos_a = jnp.cos(angles)
    sin_a = jnp.sin(angles)
    cols = jnp.arange(HIDDEN, dtype=jnp.int32)
    eq_i = cols[None, :, None] == plane_i[:, None, :]
    eq_j = cols[None, :, None] == plane_j[:, None, :]
    is_i = eq_i.any(-1)
    is_j = eq_j.any(-1)
    touched = is_i | is_j
    partner = ((eq_i * plane_j[:, None, :]).sum(-1, dtype=jnp.int32)
               + (eq_j * plane_i[:, None, :]).sum(-1, dtype=jnp.int32)
               + jnp.where(touched, 0, cols[None, :]))
    cosv = ((eq_i + eq_j) * cos_a[:, None, :]).sum(-1)
    ccos = jnp.where(touched, cosv, 1.0)
    csin = ((eq_j * sin_a[:, None, :]).sum(-1)
            - (eq_i * sin_a[:, None, :]).sum(-1))
    return partner.astype(jnp.int32), ccos, csin


def _sc_body(h_hbm, prt_hbm, ccos_hbm, csin_hbm, bias_hbm,
             out_hbm, prt_v, ccos_v, csin_v, bias_v, buf_a, buf_b):
    wid = lax.axis_index("s") * 2 + lax.axis_index("c")
    elems_per_worker = out_hbm.shape[0] // N_WORKERS
    n_tiles = elems_per_worker // TILE_ELEMS
    in_base = TC_ROWS * HIDDEN + wid * elems_per_worker
    out_base = wid * elems_per_worker

    pltpu.sync_copy(prt_hbm, prt_v)
    pltpu.sync_copy(ccos_hbm, ccos_v)
    pltpu.sync_copy(csin_hbm, csin_v)
    pltpu.sync_copy(bias_hbm, bias_v)

    def one_pass(src, dst, pp):
        def chunk_body(c, _):
            off = c * LANES
            poff = pp * HIDDEN + off
            prt = prt_v[pl.ds(poff, LANES)]
            cc = ccos_v[pl.ds(poff, LANES)]
            cs = csin_v[pl.ds(poff, LANES)]
            b = bias_v[pl.ds(poff, LANES)]

            @plsc.parallel_loop(0, TILE_ROWS, unroll=8)
            def row_body(r):
                rbase = r * HIDDEN
                h0 = src[pl.ds(rbase + off, LANES)]
                hp = plsc.load_gather(src, [prt + rbase])
                z = cc * h0 + (cs * hp + b)
                dst[pl.ds(rbase + off, LANES)] = z / (1.0 + jnp.exp(-z))

            return 0

        lax.fori_loop(0, HIDDEN // LANES, chunk_body, 0)

    def tile_body(t, _):
        pltpu.sync_copy(h_hbm.at[pl.ds(in_base + t * TILE_ELEMS, TILE_ELEMS)],
                        buf_a)
        one_pass(buf_a, buf_b, 0)
        one_pass(buf_b, buf_a, 1)
        one_pass(buf_a, buf_b, 2)
        pltpu.sync_copy(buf_b,
                        out_hbm.at[pl.ds(out_base + t * TILE_ELEMS, TILE_ELEMS)])
        return 0

    lax.fori_loop(0, n_tiles, tile_body, 0)


def _sc_part(h_full_flat, n_sc_rows, partner, ccos_g, csin_g, bias):
    mesh = plsc.VectorSubcoreMesh(core_axis_name="c", subcore_axis_name="s")
    sc_kernel = functools.partial(
        pl.kernel,
        out_type=jax.ShapeDtypeStruct((n_sc_rows * HIDDEN,), jnp.float32),
        mesh=mesh,
        compiler_params=pltpu.CompilerParams(needs_layout_passes=False),
        scratch_types=[
            pltpu.VMEM((N_PASSES * HIDDEN,), jnp.int32),
            pltpu.VMEM((N_PASSES * HIDDEN,), jnp.float32),
            pltpu.VMEM((N_PASSES * HIDDEN,), jnp.float32),
            pltpu.VMEM((N_PASSES * HIDDEN,), jnp.float32),
            pltpu.VMEM((TILE_ELEMS,), jnp.float32),
            pltpu.VMEM((TILE_ELEMS,), jnp.float32),
        ],
    )(_sc_body)
    return sc_kernel(h_full_flat, partner.reshape(-1), ccos_g.reshape(-1),
                     csin_g.reshape(-1), bias.reshape(-1))


def _tc_body(x_ref, S_ref, dcosg_ref, bias_ref, o_ref):
    h = x_ref[...]
    for pp in range(N_PASSES):
        hp = jnp.dot(h.astype(jnp.bfloat16), S_ref[pp],
                     preferred_element_type=jnp.float32)
        z = h * dcosg_ref[pp][None, :] + (hp + bias_ref[pp][None, :])
        h = z / (1.0 + jnp.exp(-z))
    o_ref[...] = h


def _tc_body_b(prev_ref, x_ref, S_ref, dcosg_ref, bias_ref, o_ref):
    del prev_ref
    _tc_body(x_ref, S_ref, dcosg_ref, bias_ref, o_ref)


def _tc_part(h_full, partner, csin_g, dcosg, bias):
    n_rows = h_full.shape[0]
    cols = jnp.arange(HIDDEN, dtype=jnp.int32)
    S = jnp.where(cols[None, :, None] == partner[:, None, :],
                  csin_g[:, None, :], 0.0).astype(jnp.bfloat16)
    tbl_specs = [
        pl.BlockSpec((N_PASSES, HIDDEN, HIDDEN), lambda i: (0, 0, 0)),
        pl.BlockSpec((N_PASSES, HIDDEN), lambda i: (0, 0)),
        pl.BlockSpec((N_PASSES, HIDDEN), lambda i: (0, 0)),
    ]
    out_a = pl.pallas_call(
        _tc_body,
        grid=(TC_A_ROWS // TC_BLOCK_ROWS,),
        in_specs=[pl.BlockSpec((TC_BLOCK_ROWS, HIDDEN), lambda i: (i, 0))]
        + tbl_specs,
        out_specs=pl.BlockSpec((TC_BLOCK_ROWS, HIDDEN), lambda i: (i, 0)),
        out_shape=jax.ShapeDtypeStruct((n_rows, HIDDEN), jnp.float32),
    )(h_full, S, dcosg, bias)
    off = TC_A_ROWS // TC_BLOCK_ROWS
    return pl.pallas_call(
        _tc_body_b,
        grid=((TC_ROWS - TC_A_ROWS) // TC_BLOCK_ROWS,),
        in_specs=[
            pl.BlockSpec((TC_BLOCK_ROWS, HIDDEN), lambda i: (i + off, 0)),
            pl.BlockSpec((TC_BLOCK_ROWS, HIDDEN), lambda i: (i + off, 0)),
        ] + tbl_specs,
        out_specs=pl.BlockSpec((TC_BLOCK_ROWS, HIDDEN), lambda i: (i + off, 0)),
        out_shape=jax.ShapeDtypeStruct((n_rows, HIDDEN), jnp.float32),
        input_output_aliases={0: 0},
    )(out_a, h_full, S, dcosg, bias)


def kernel(x, angles, gate, bias, plane_i, plane_j):
    orig_shape = x.shape
    n_rows = x.size // HIDDEN
    h = x.reshape(n_rows, HIDDEN)
    n_sc_rows = n_rows - TC_ROWS
    partner, ccos, csin = _col_params(angles, plane_i, plane_j)
    ccos_g = ccos * gate
    csin_g = csin * gate

    tc_out = _tc_part(h, partner, csin_g, ccos_g, bias)
    sc_out = _sc_part(h.reshape(-1), n_sc_rows, partner, ccos_g, csin_g, bias)
    out = lax.dynamic_update_slice(tc_out, sc_out.reshape(-1, HIDDEN),
                                   (TC_ROWS, 0))
    return out.reshape(orig_shape)


# speedup vs baseline: 1.1927x; 1.1927x over previous
import functools

import jax
import jax.numpy as jnp
from jax import lax
from jax.experimental import pallas as pl
from jax.experimental.pallas import tpu as pltpu
from jax.experimental.pallas import tpu_sc as plsc

HIDDEN = 1024
N_PASSES = 3
LANES = 16
N_WORKERS = 32
TILE_ROWS = 32
TILE_ELEMS = TILE_ROWS * HIDDEN

TC_ROWS = 25600
TC_BLOCK_ROWS = 1024


def _col_params(angles, plane_i, plane_j):
    cos_a = jnp.cos(angles)
    sin_a = jnp.sin(angles)
    cols = jnp.arange(HIDDEN, dtype=jnp.int32)
    eq_i = cols[None, :, None] == plane_i[:, None, :]
    eq_j = cols[None, :, None] == plane_j[:, None, :]
    is_i = eq_i.any(-1)
    is_j = eq_j.any(-1)
    touched = is_i | is_j
    partner = ((eq_i * plane_j[:, None, :]).sum(-1, dtype=jnp.int32)
               + (eq_j * plane_i[:, None, :]).sum(-1, dtype=jnp.int32)
               + jnp.where(touched, 0, cols[None, :]))
    cosv = ((eq_i + eq_j) * cos_a[:, None, :]).sum(-1)
    ccos = jnp.where(touched, cosv, 1.0)
    csin = ((eq_j * sin_a[:, None, :]).sum(-1)
            - (eq_i * sin_a[:, None, :]).sum(-1))
    return partner.astype(jnp.int32), ccos, csin


def _sc_body(h_hbm, prt_hbm, ccos_hbm, csin_hbm, bias_hbm,
             out_hbm, prt_v, ccos_v, csin_v, bias_v, buf_a, buf_b):
    wid = lax.axis_index("s") * 2 + lax.axis_index("c")
    elems_per_worker = out_hbm.shape[0] // N_WORKERS
    n_tiles = elems_per_worker // TILE_ELEMS
    in_base = wid * elems_per_worker
    out_base = wid * elems_per_worker

    pltpu.sync_copy(prt_hbm, prt_v)
    pltpu.sync_copy(ccos_hbm, ccos_v)
    pltpu.sync_copy(csin_hbm, csin_v)
    pltpu.sync_copy(bias_hbm, bias_v)

    def one_pass(src, dst, pp):
        def chunk_body(c, _):
            off = c * LANES
            poff = pp * HIDDEN + off
            prt = prt_v[pl.ds(poff, LANES)]
            cc = ccos_v[pl.ds(poff, LANES)]
            cs = csin_v[pl.ds(poff, LANES)]
            b = bias_v[pl.ds(poff, LANES)]

            @plsc.parallel_loop(0, TILE_ROWS, unroll=8)
            def row_body(r):
                rbase = r * HIDDEN
                h0 = src[pl.ds(rbase + off, LANES)]
                hp = plsc.load_gather(src, [prt + rbase])
                z = cc * h0 + (cs * hp + b)
                dst[pl.ds(rbase + off, LANES)] = z / (1.0 + jnp.exp(-z))

            return 0

        lax.fori_loop(0, HIDDEN // LANES, chunk_body, 0)

    def tile_body(t, _):
        pltpu.sync_copy(h_hbm.at[pl.ds(in_base + t * TILE_ELEMS, TILE_ELEMS)],
                        buf_a)
        one_pass(buf_a, buf_b, 0)
        one_pass(buf_b, buf_a, 1)
        one_pass(buf_a, buf_b, 2)
        pltpu.sync_copy(buf_b,
                        out_hbm.at[pl.ds(out_base + t * TILE_ELEMS, TILE_ELEMS)])
        return 0

    lax.fori_loop(0, n_tiles, tile_body, 0)


def _sc_part(h_full_flat, n_sc_rows, partner, ccos_g, csin_g, bias):
    mesh = plsc.VectorSubcoreMesh(core_axis_name="c", subcore_axis_name="s")
    sc_kernel = functools.partial(
        pl.kernel,
        out_type=jax.ShapeDtypeStruct((n_sc_rows * HIDDEN,), jnp.float32),
        mesh=mesh,
        compiler_params=pltpu.CompilerParams(needs_layout_passes=False),
        scratch_types=[
            pltpu.VMEM((N_PASSES * HIDDEN,), jnp.int32),
            pltpu.VMEM((N_PASSES * HIDDEN,), jnp.float32),
            pltpu.VMEM((N_PASSES * HIDDEN,), jnp.float32),
            pltpu.VMEM((N_PASSES * HIDDEN,), jnp.float32),
            pltpu.VMEM((TILE_ELEMS,), jnp.float32),
            pltpu.VMEM((TILE_ELEMS,), jnp.float32),
        ],
    )(_sc_body)
    return sc_kernel(h_full_flat, partner.reshape(-1), ccos_g.reshape(-1),
                     csin_g.reshape(-1), bias.reshape(-1))


def _tc_body(x_ref, S_ref, dcosg_ref, bias_ref, o_ref):
    h = x_ref[...]
    for pp in range(N_PASSES):
        hp = jnp.dot(h.astype(jnp.bfloat16), S_ref[pp],
                     preferred_element_type=jnp.float32)
        z = h * dcosg_ref[pp][None, :] + (hp + bias_ref[pp][None, :])
        h = z / (1.0 + jnp.exp(-z))
    o_ref[...] = h


def _tc_part(h_full, partner, csin_g, dcosg, bias):
    n_rows = h_full.shape[0]
    cols = jnp.arange(HIDDEN, dtype=jnp.int32)
    S = jnp.where(cols[None, :, None] == partner[:, None, :],
                  csin_g[:, None, :], 0.0).astype(jnp.bfloat16)
    return pl.pallas_call(
        _tc_body,
        grid=(TC_ROWS // TC_BLOCK_ROWS,),
        in_specs=[
            pl.BlockSpec((TC_BLOCK_ROWS, HIDDEN), lambda i: (i, 0)),
            pl.BlockSpec((N_PASSES, HIDDEN, HIDDEN), lambda i: (0, 0, 0)),
            pl.BlockSpec((N_PASSES, HIDDEN), lambda i: (0, 0)),
            pl.BlockSpec((N_PASSES, HIDDEN), lambda i: (0, 0)),
        ],
        out_specs=pl.BlockSpec((TC_BLOCK_ROWS, HIDDEN), lambda i: (i, 0)),
        out_shape=jax.ShapeDtypeStruct((n_rows, HIDDEN), jnp.float32),
    )(h_full, S, dcosg, bias)


def kernel(x, angles, gate, bias, plane_i, plane_j):
    orig_shape = x.shape
    n_rows = x.size // HIDDEN
    h = x.reshape(n_rows, HIDDEN)
    n_sc_rows = n_rows - TC_ROWS
    partner, ccos, csin = _col_params(angles, plane_i, plane_j)
    ccos_g = ccos * gate
    csin_g = csin * gate

    tc_out = _tc_part(h, partner, csin_g, ccos_g, bias)
    sc_out = _sc_part(h[TC_ROWS:].reshape(-1), n_sc_rows, partner, ccos_g,
                      csin_g, bias)
    out = lax.dynamic_update_slice(tc_out, sc_out.reshape(-1, HIDDEN),
                                   (TC_ROWS, 0))
    return out.reshape(orig_shape)
